# baseline (speedup 1.0000x reference)
import jax
import jax.numpy as jnp
from jax import lax
from jax.experimental import pallas as pl
from jax.experimental.pallas import tpu as pltpu

_B = 16
_N = 20000
_K = 2000
_KP = 2048
_T = 128
_NT = _KP // _T
_OUT = 1000
_OUTP = 1024
_IOU = 0.7


def _iou(ry1, rx1, ry2, rx2, rarea, cy1, cx1, cy2, cx2, carea):
    yy1 = jnp.maximum(ry1, cy1)
    xx1 = jnp.maximum(rx1, cx1)
    yy2 = jnp.minimum(ry2, cy2)
    xx2 = jnp.minimum(rx2, cx2)
    inter = jnp.maximum(yy2 - yy1, 0.0) * jnp.maximum(xx2 - xx1, 0.0)
    union = rarea + carea - inter
    return inter / jnp.maximum(union, 1e-8)


def _nms_body(boxes_ref, scores_ref, data_ref, out_ref, act_ref):
    bx = boxes_ref[0]
    sc = scores_ref[0]
    y1, x1, y2, x2 = bx[0], bx[1], bx[2], bx[3]
    area = (y2 - y1) * (x2 - x1)
    init = (sc > 0.0).astype(jnp.float32)

    act_ref[...] = jnp.zeros((16, 128), jnp.float32)

    ii = lax.broadcasted_iota(jnp.int32, (_T, _T), 0)
    jj = lax.broadcasted_iota(jnp.int32, (_T, _T), 1)
    tri = (ii < jj).astype(jnp.float32)

    def tile_step(t, _):
        tb = boxes_ref[0, :, pl.ds(t, 1), :]
        y1t, x1t, y2t, x2t = tb[0], tb[1], tb[2], tb[3]
        at = (y2t - y1t) * (x2t - x1t)
        itile = (scores_ref[0, pl.ds(t, 1), :] > 0.0).astype(jnp.float32)

        active = act_ref[...]

        m_full = _iou(y1[:, :, None], x1[:, :, None],
                      y2[:, :, None], x2[:, :, None], area[:, :, None],
                      y1t[None], x1t[None], y2t[None], x2t[None],
                      at[None]) > _IOU
        cross = jnp.max(jnp.max(jnp.where(m_full, active[:, :, None], 0.0),
                                axis=0), axis=0, keepdims=True)
        a0 = itile * (1.0 - cross)

        m_tt = (_iou(y1t.T, x1t.T, y2t.T, x2t.T, at.T,
                     y1t, x1t, y2t, x2t, at) > _IOU
                ).astype(jnp.float32)
        e0 = m_tt * tri * a0 * a0.T

        def w_cond(carry):
            return carry[1]

        def w_body(carry):
            e, _ = carry
            inc = jnp.max(e, axis=0, keepdims=True)
            dead = jnp.max(e * (1.0 - inc).T, axis=0, keepdims=True)
            e2 = e * (1.0 - dead).T
            return e2, jnp.sum(e2) < jnp.sum(e)

        e_fin, _ = lax.while_loop(w_cond, w_body, (e0, jnp.sum(e0) > 0.0))
        suppressed = jnp.max(e_fin, axis=0, keepdims=True)
        act_ref[pl.ds(t, 1), :] = a0 * (1.0 - suppressed)
        return 0

    lax.fori_loop(0, _NT, tile_step, 0)

    act = act_ref[...]
    pr = lax.dot_general(act, (jj < ii).astype(jnp.float32),
                         (((1,), (1,)), ((), ())),
                         preferred_element_type=jnp.float32,
                         precision=lax.Precision.HIGHEST)
    rt = pr[:, 127:128] + act[:, 127:128]
    i16 = lax.broadcasted_iota(jnp.int32, (16, 16), 0)
    j16 = lax.broadcasted_iota(jnp.int32, (16, 16), 1)
    offs = jnp.sum(jnp.where(j16 < i16, rt.T, 0.0), axis=1,
                   keepdims=True)
    pos = pr + offs

    siota = lax.broadcasted_iota(jnp.int32, (_OUTP, _T), 0)
    posi = pos.astype(jnp.int32)
    acc = jnp.zeros((8, _OUTP), jnp.float32)
    for r in range(16):
        oh = jnp.where(siota == posi[r:r + 1, :],
                       act[r:r + 1, :], 0.0)
        acc = acc + lax.dot_general(
            data_ref[0, :, r, :], oh, (((1,), (1,)), ((), ())),
            preferred_element_type=jnp.float32,
            precision=lax.Precision.HIGHEST)
    out_ref[0] = acc


def kernel(multi_level_boxes, multi_level_scores):
    top_scores, idx = lax.top_k(multi_level_scores, _K)
    top_boxes = jnp.take_along_axis(multi_level_boxes, idx[:, :, None],
                                    axis=1)

    tb = jnp.pad(top_boxes, ((0, 0), (0, _KP - _K), (0, 0)))
    ts = jnp.pad(top_scores, ((0, 0), (0, _KP - _K)),
                 constant_values=-1.0)
    tbt = tb.transpose(0, 2, 1).reshape(_B, 4, _NT, 128)
    tsr = ts.reshape(_B, _NT, 128)
    data_all = jnp.concatenate(
        [tbt, tsr[:, None], jnp.zeros((_B, 3, _NT, 128), jnp.float32)],
        axis=1)

    out = pl.pallas_call(
        _nms_body,
        grid=(_B,),
        in_specs=[
            pl.BlockSpec((1, 4, _NT, 128), lambda b: (b, 0, 0, 0)),
            pl.BlockSpec((1, _NT, 128), lambda b: (b, 0, 0)),
            pl.BlockSpec((1, 8, _NT, 128), lambda b: (b, 0, 0, 0)),
        ],
        out_specs=pl.BlockSpec((1, 8, _OUTP), lambda b: (b, 0, 0)),
        out_shape=jax.ShapeDtypeStruct((_B, 8, _OUTP), jnp.float32),
        scratch_shapes=[pltpu.VMEM((16, 128), jnp.float32)],
    )(tbt, tsr, data_all)

    rois = out[:, 0:4, :_OUT].transpose(0, 2, 1)
    rscores = out[:, 4, :_OUT]
    return rois, rscores


# speedup vs baseline: 1.5777x; 1.5777x over previous
import jax
import jax.numpy as jnp
from jax import lax
from jax.experimental import pallas as pl
from jax.experimental.pallas import tpu as pltpu

_B = 16
_N = 20000
_K = 2000
_KP = 2048
_T = 128
_NT = _KP // _T
_OUT = 1000
_OUTP = 1024
_IOU = 0.7


def _iou(ry1, rx1, ry2, rx2, rarea, cy1, cx1, cy2, cx2, carea):
    yy1 = jnp.maximum(ry1, cy1)
    xx1 = jnp.maximum(rx1, cx1)
    yy2 = jnp.minimum(ry2, cy2)
    xx2 = jnp.minimum(rx2, cx2)
    inter = jnp.maximum(yy2 - yy1, 0.0) * jnp.maximum(xx2 - xx1, 0.0)
    union = rarea + carea - inter
    return inter / jnp.maximum(union, 1e-8)


def _matvec(v, m):
    return lax.dot_general(v, m, (((1,), (0,)), ((), ())),
                           preferred_element_type=jnp.float32)


def _nms_body(boxes_ref, scores_ref, data_ref, out_ref):
    bx = boxes_ref[0]
    sc = scores_ref[0]
    y1, x1, y2, x2 = bx[0], bx[1], bx[2], bx[3]
    area = (y2 - y1) * (x2 - x1)
    init = (sc > 0.0).astype(jnp.float32)

    ii = lax.broadcasted_iota(jnp.int32, (_T, _T), 0)
    jj = lax.broadcasted_iota(jnp.int32, (_T, _T), 1)
    tri = (ii < jj).astype(jnp.float32)

    acts = []
    for t in range(_NT):
        y1t, x1t = y1[t:t + 1], x1[t:t + 1]
        y2t, x2t = y2[t:t + 1], x2[t:t + 1]
        at = area[t:t + 1]
        itile = init[t:t + 1]

        if t > 0:
            m_prev = _iou(y1[:t, :, None], x1[:t, :, None],
                          y2[:t, :, None], x2[:t, :, None],
                          area[:t, :, None],
                          y1t[None], x1t[None], y2t[None], x2t[None],
                          at[None]) > _IOU
            aprev = jnp.concatenate(acts, axis=0)
            cross = jnp.max(jnp.max(jnp.where(m_prev, aprev[:, :, None],
                                              0.0), axis=0),
                            axis=0, keepdims=True)
            a0 = itile * (1.0 - cross)
        else:
            a0 = itile

        m_tt = (_iou(y1t.T, x1t.T, y2t.T, x2t.T, at.T,
                     y1t, x1t, y2t, x2t, at) > _IOU
                ).astype(jnp.float32)
        e0 = m_tt * tri * a0 * a0.T

        def w_cond(carry):
            return carry[1]

        def w_body(carry):
            r, _ = carry
            inc = _matvec(1.0 - r, e0)
            cansup = (inc == 0.0).astype(jnp.float32)
            dead = (_matvec(cansup, e0) > 0.0).astype(jnp.float32)
            r2 = jnp.maximum(r, dead)
            return r2, jnp.sum(r2) > jnp.sum(r)

        r0 = jnp.zeros((1, _T), jnp.float32)
        r_fin, _ = lax.while_loop(w_cond, w_body,
                                  (r0, jnp.sum(e0) > 0.0))
        suppressed = (_matvec(1.0 - r_fin, e0) > 0.0).astype(jnp.float32)
        acts.append(a0 * (1.0 - suppressed))

    act = jnp.concatenate(acts, axis=0)

    pr = lax.dot_general(act, (jj < ii).astype(jnp.float32),
                         (((1,), (1,)), ((), ())),
                         preferred_element_type=jnp.float32,
                         precision=lax.Precision.HIGHEST)
    rt = pr[:, 127:128] + act[:, 127:128]
    i16 = lax.broadcasted_iota(jnp.int32, (16, 16), 0)
    j16 = lax.broadcasted_iota(jnp.int32, (16, 16), 1)
    offs = jnp.sum(jnp.where(j16 < i16, rt.T, 0.0), axis=1,
                   keepdims=True)
    pos = pr + offs

    siota = lax.broadcasted_iota(jnp.int32, (_OUTP, _T), 0)
    posi = pos.astype(jnp.int32)
    acc = jnp.zeros((8, _OUTP), jnp.float32)
    for r in range(16):
        oh = jnp.where(siota == posi[r:r + 1, :],
                       act[r:r + 1, :], 0.0)
        acc = acc + lax.dot_general(
            data_ref[0, :, r, :], oh, (((1,), (1,)), ((), ())),
            preferred_element_type=jnp.float32,
            precision=lax.Precision.HIGHEST)
    out_ref[0] = acc


def kernel(multi_level_boxes, multi_level_scores):
    top_scores, idx = lax.top_k(multi_level_scores, _K)
    top_boxes = jnp.take_along_axis(multi_level_boxes, idx[:, :, None],
                                    axis=1)

    tb = jnp.pad(top_boxes, ((0, 0), (0, _KP - _K), (0, 0)))
    ts = jnp.pad(top_scores, ((0, 0), (0, _KP - _K)),
                 constant_values=-1.0)
    tbt = tb.transpose(0, 2, 1).reshape(_B, 4, _NT, 128)
    tsr = ts.reshape(_B, _NT, 128)
    data_all = jnp.concatenate(
        [tbt, tsr[:, None], jnp.zeros((_B, 3, _NT, 128), jnp.float32)],
        axis=1)

    out = pl.pallas_call(
        _nms_body,
        grid=(_B,),
        in_specs=[
            pl.BlockSpec((1, 4, _NT, 128), lambda b: (b, 0, 0, 0)),
            pl.BlockSpec((1, _NT, 128), lambda b: (b, 0, 0)),
            pl.BlockSpec((1, 8, _NT, 128), lambda b: (b, 0, 0, 0)),
        ],
        out_specs=pl.BlockSpec((1, 8, _OUTP), lambda b: (b, 0, 0)),
        out_shape=jax.ShapeDtypeStruct((_B, 8, _OUTP), jnp.float32),
    )(tbt, tsr, data_all)

    rois = out[:, 0:4, :_OUT].transpose(0, 2, 1)
    rscores = out[:, 4, :_OUT]
    return rois, rscores
